# baseline (speedup 1.0000x reference)
import functools

import jax
import jax.numpy as jnp
from jax.experimental import pallas as pl
from jax.experimental.pallas import tpu as pltpu

_B = 1024
_K = 256
_N = 100000
_CHOOSE = 256
_INV_TEMP = 40.0

_NB = 512
_NPAD = ((_N + _NB - 1) // _NB) * _NB


def _mm_body(q_ref, nq_ref, k_ref, nk_ref, out_ref, qn_ref):
    i = pl.program_id(0)

    @pl.when(i == 0)
    def _():
        qn_ref[...] = q_ref[...] / nq_ref[...]

    kn = k_ref[...] / nk_ref[...]
    out_ref[...] = jax.lax.dot_general(
        qn_ref[...], kn, (((1,), (1,)), ((), ())),
        preferred_element_type=jnp.float32,
        precision=jax.lax.Precision.DEFAULT)


_sims_call = pl.pallas_call(
    _mm_body,
    grid=(_NPAD // _NB,),
    in_specs=[
        pl.BlockSpec((_B, _K), lambda i: (0, 0)),
        pl.BlockSpec((_B, 1), lambda i: (0, 0)),
        pl.BlockSpec((_NB, _K), lambda i: (i, 0)),
        pl.BlockSpec((_NB, 1), lambda i: (i, 0)),
    ],
    out_specs=pl.BlockSpec((_B, _NB), lambda i: (0, i)),
    out_shape=jax.ShapeDtypeStruct((_B, _NPAD), jnp.float32),
    scratch_shapes=[pltpu.VMEM((_B, _K), jnp.float32)],
)


def kernel(input, keys, value):
    nq = jnp.linalg.norm(input, axis=-1, keepdims=True) + 1e-8
    nk = jnp.linalg.norm(keys, axis=-1, keepdims=True) + 1e-8
    keys_pad = jnp.pad(keys, ((0, _NPAD - _N), (0, 0)))
    nk_pad = jnp.pad(nk, ((0, _NPAD - _N), (0, 0)), constant_values=1.0)
    sims = _sims_call(input, nq, keys_pad, nk_pad)[:, :_N]
    topk_sims, topk_idx = jax.lax.top_k(sims, _CHOOSE)
    score = jax.nn.softmax(_INV_TEMP * topk_sims, axis=-1)
    vals = jnp.take(value, topk_idx, axis=0)
    y = jnp.sum(score * vals, axis=-1)
    return y, topk_sims, topk_idx


# speedup vs baseline: 6.2893x; 6.2893x over previous
import functools

import jax
import jax.numpy as jnp
from jax import lax
from jax.experimental import pallas as pl
from jax.experimental.pallas import tpu as pltpu
from jax.experimental.pallas import tpu_sc as plsc

_B = 1024
_K = 256
_N = 100000
_CHOOSE = 256
_INV_TEMP = 40.0


_NB = 1024
_NPAD = ((_N + _NB - 1) // _NB) * _NB
_SLAB = _NPAD // 128


def _mm_body(q_ref, nq_ref, k_ref, nk_ref, out_ref, qn_ref):
    i = pl.program_id(0)

    @pl.when(i == 0)
    def _():
        qn_ref[...] = q_ref[...] / nq_ref[...]

    kn = k_ref[...] / nk_ref[...]
    for t in range(_NB // 128):
        out_ref[:, t, :] = jax.lax.dot_general(
            qn_ref[...], kn[t * 128:(t + 1) * 128, :],
            (((1,), (1,)), ((), ())),
            preferred_element_type=jnp.float32,
            precision=jax.lax.Precision.DEFAULT)


_sims_call = pl.pallas_call(
    _mm_body,
    grid=(_NPAD // _NB,),
    in_specs=[
        pl.BlockSpec((_B, _K), lambda i: (0, 0)),
        pl.BlockSpec((_B, 1), lambda i: (0, 0)),
        pl.BlockSpec((_NB, _K), lambda i: (i, 0)),
        pl.BlockSpec((_NB, 1), lambda i: (i, 0)),
    ],
    out_specs=pl.BlockSpec((_B, _NB // 128, 128), lambda i: (0, i, 0)),
    out_shape=jax.ShapeDtypeStruct((_B, _SLAB, 128), jnp.float32),
    scratch_shapes=[pltpu.VMEM((_B, _K), jnp.float32)],
)


_L = 16
_NW = 32
_ROWS_PER_W = _B // _NW
_VREGS_ROW = _N // _L
_NBINS = 1024
_LO = -1.03125
_SCALE = 496.0
_CAP = 512
_CVR = _CAP // _L


def _sc_body(sims_hbm, value_hbm, out_sims, out_idx, out_y,
             row_v, hist_v, cand_val, cand_idx, osims_v, oidx_v, vals_v,
             ybuf, sem):
    wid = lax.axis_index("s") * 2 + lax.axis_index("c")
    lanes = lax.iota(jnp.int32, _L)
    ones_i = jnp.ones((_L,), jnp.int32)
    zeros_i = jnp.zeros((_L,), jnp.int32)
    neg2 = jnp.full((_L,), -2.0, jnp.float32)

    def rload(v):
        s = v // 8
        return row_v[s, pl.ds((v - s * 8) * _L, _L)]

    def cval(v):
        return cand_val[pl.ds(v * _L, _L)]

    def cidx(v):
        return cand_idx[pl.ds(v * _L, _L)]

    @pl.loop(0, _ROWS_PER_W)
    def _row(rl):
        r = wid * _ROWS_PER_W + rl
        pltpu.sync_copy(sims_hbm.at[r], row_v)

        @pl.loop(0, _NBINS)
        def _zh(i):
            hist_v[pl.ds(i * _L, _L)] = zeros_i

        @pl.loop(0, _VREGS_ROW)
        def _pa(j):
            v = rload(j)
            b = jnp.clip(((v - _LO) * _SCALE).astype(jnp.int32), 0, _NBINS - 1)
            plsc.addupdate_scatter(hist_v, [b * _L + lanes], ones_i)

        def _wcond(carry):
            cum, b = carry
            return jnp.logical_and(cum < _CHOOSE, b >= 0)

        def _wstep(carry):
            cum, b = carry
            h = hist_v[pl.ds(b * _L, _L)]
            return cum + jnp.sum(h), b - 1

        _, bend = lax.while_loop(_wcond, _wstep,
                                 (jnp.int32(0), jnp.int32(_NBINS - 1)))
        bstar = bend + 1

        @pl.loop(0, _CVR)
        def _zc(i):
            cand_val[pl.ds(i * _L, _L)] = neg2
            cand_idx[pl.ds(i * _L, _L)] = zeros_i

        def _pb(j, off):
            v = rload(j)
            b = jnp.clip(((v - _LO) * _SCALE).astype(jnp.int32), 0, _NBINS - 1)
            m = b >= bstar
            mi = m.astype(jnp.int32)
            cnt = jnp.sum(mi)

            @pl.when(cnt > 0)
            def _():
                pos = off + plsc.cumsum(mi) - 1
                g = jnp.logical_and(m, pos < _CAP)
                plsc.store_scatter(cand_val, [pos], v, mask=g)
                plsc.store_scatter(cand_idx, [pos], j * _L + lanes, mask=g)

            return off + cnt

        lax.fori_loop(0, _VREGS_ROW, _pb, jnp.int32(0))

        def _ce(a, b, kv):
            ka = cval(a)
            kb = cval(b)
            ia = cidx(a)
            ib = cidx(b)
            desc = jnp.broadcast_to((a & kv) == 0, (_L,))
            swap = jnp.where(desc, ka < kb, ka > kb)
            cand_val[pl.ds(a * _L, _L)] = jnp.where(swap, kb, ka)
            cand_val[pl.ds(b * _L, _L)] = jnp.where(swap, ka, kb)
            cand_idx[pl.ds(a * _L, _L)] = jnp.where(swap, ib, ia)
            cand_idx[pl.ds(b * _L, _L)] = jnp.where(swap, ia, ib)

        def _vsort(v, desc):
            ks, xs = plsc.sort_key_val(cval(v), cidx(v), descending=desc)
            cand_val[pl.ds(v * _L, _L)] = ks
            cand_idx[pl.ds(v * _L, _L)] = xs

        @pl.loop(0, _CVR // 2)
        def _base(t):
            _vsort(2 * t, True)
            _vsort(2 * t + 1, False)

        for kv in (2, 4, 8, 16, 32):
            jv = kv // 2
            while jv >= 1:
                @pl.loop(0, _CVR // 2)
                def _stage(t, jv=jv, kv=kv):
                    blk = t // jv
                    a = blk * (2 * jv) + (t - blk * jv)
                    _ce(a, a + jv, kv)
                jv //= 2
            if kv < _CVR:
                @pl.loop(0, _CVR // 2)
                def _resort(t, kv=kv):
                    blk = t // kv
                    v = blk * (2 * kv) + (t - blk * kv)
                    _vsort(v, True)
                    _vsort(v + kv, False)
            else:
                @pl.loop(0, _CVR)
                def _resort_all(v):
                    _vsort(v, True)

        @pl.loop(0, _CHOOSE // _L)
        def _st(t):
            s = t // 8
            c = (t - s * 8) * _L
            osims_v[s, pl.ds(c, _L)] = cval(t)
            oidx_v[s, pl.ds(c, _L)] = cidx(t)

        pltpu.sync_copy(osims_v, out_sims.at[r])
        pltpu.sync_copy(oidx_v, out_idx.at[r])

        cp0 = pltpu.async_copy(value_hbm.at[oidx_v.at[0]], vals_v.at[0], sem)
        cp0.wait()
        cp1 = pltpu.async_copy(value_hbm.at[oidx_v.at[1]], vals_v.at[1], sem)
        cp1.wait()

        mx = jnp.max(cval(0))

        def _sm(t, carry):
            accn, accd = carry
            s = t // 8
            c = (t - s * 8) * _L
            e = jnp.exp((osims_v[s, pl.ds(c, _L)] - mx) * _INV_TEMP)
            return accn + e * vals_v[s, pl.ds(c, _L)], accd + e

        accn, accd = lax.fori_loop(
            0, _CHOOSE // _L, _sm,
            (jnp.zeros((_L,), jnp.float32), jnp.zeros((_L,), jnp.float32)))
        yv = (jnp.broadcast_to(jnp.sum(accn), (_L,))
              / jnp.broadcast_to(jnp.sum(accd), (_L,)))
        plsc.store_scatter(ybuf, [jnp.broadcast_to(rl, (_L,))],
                           yv, mask=lanes < 1)

    pltpu.sync_copy(ybuf, out_y.at[pl.ds(wid * _ROWS_PER_W, _ROWS_PER_W)])


_sc_select = pl.kernel(
    _sc_body,
    out_type=(
        jax.ShapeDtypeStruct((_B, 2, 128), jnp.float32),
        jax.ShapeDtypeStruct((_B, 2, 128), jnp.int32),
        jax.ShapeDtypeStruct((_B,), jnp.float32),
    ),
    mesh=plsc.VectorSubcoreMesh(core_axis_name="c", subcore_axis_name="s"),
    compiler_params=pltpu.CompilerParams(needs_layout_passes=False),
    scratch_types=[
        pltpu.VMEM((_SLAB, 128), jnp.float32),
        pltpu.VMEM((_NBINS * _L,), jnp.int32),
        pltpu.VMEM((_CAP,), jnp.float32),
        pltpu.VMEM((_CAP,), jnp.int32),
        pltpu.VMEM((2, 128), jnp.float32),
        pltpu.VMEM((2, 128), jnp.int32),
        pltpu.VMEM((2, 128), jnp.float32),
        pltpu.VMEM((_ROWS_PER_W,), jnp.float32),
        pltpu.SemaphoreType.DMA,
    ],
)


def kernel(input, keys, value):
    nq = jnp.linalg.norm(input, axis=-1, keepdims=True) + 1e-8
    nk = jnp.linalg.norm(keys, axis=-1, keepdims=True) + 1e-8
    keys_pad = jnp.pad(keys, ((0, _NPAD - _N), (0, 0)))
    nk_pad = jnp.pad(nk, ((0, _NPAD - _N), (0, 0)), constant_values=1.0)
    sims = _sims_call(input, nq, keys_pad, nk_pad)
    topk_sims, topk_idx, y = _sc_select(sims, value)
    return (y, topk_sims.reshape(_B, _CHOOSE), topk_idx.reshape(_B, _CHOOSE))


# speedup vs baseline: 12.5936x; 2.0024x over previous
import functools

import jax
import jax.numpy as jnp
from jax import lax
from jax.experimental import pallas as pl
from jax.experimental.pallas import tpu as pltpu
from jax.experimental.pallas import tpu_sc as plsc

_B = 1024
_K = 256
_N = 100000
_CHOOSE = 256
_INV_TEMP = 40.0


_NB = 1024
_NPAD = ((_N + _NB - 1) // _NB) * _NB
_SLAB = _NPAD // 128


def _mm_body(q_ref, nq_ref, k_ref, nk_ref, out_ref, qn_ref):
    i = pl.program_id(0)

    @pl.when(i == 0)
    def _():
        qn_ref[...] = q_ref[...] / nq_ref[...]

    kn = k_ref[...] / nk_ref[...]
    for t in range(_NB // 128):
        out_ref[:, t, :] = jax.lax.dot_general(
            qn_ref[...], kn[t * 128:(t + 1) * 128, :],
            (((1,), (1,)), ((), ())),
            preferred_element_type=jnp.float32,
            precision=jax.lax.Precision.DEFAULT)


_sims_call = pl.pallas_call(
    _mm_body,
    grid=(_NPAD // _NB,),
    in_specs=[
        pl.BlockSpec((_B, _K), lambda i: (0, 0)),
        pl.BlockSpec((_B, 1), lambda i: (0, 0)),
        pl.BlockSpec((_NB, _K), lambda i: (i, 0)),
        pl.BlockSpec((_NB, 1), lambda i: (i, 0)),
    ],
    out_specs=pl.BlockSpec((_B, _NB // 128, 128), lambda i: (0, i, 0)),
    out_shape=jax.ShapeDtypeStruct((_B, _SLAB, 128), jnp.float32),
    scratch_shapes=[pltpu.VMEM((_B, _K), jnp.float32)],
)


_L = 16
_NW = 32
_ROWS_PER_W = _B // _NW
_VREGS_ROW = _N // _L
_NBINS = 1024
_LO = -1.03125
_SCALE = 496.0
_CAP = 512
_CVR = _CAP // _L


def _sc_body(sims_hbm, value_hbm, out_sims, out_idx, out_y,
             row_v, hist_v, binsum_v, cand_val, cand_idx, osims_v, oidx_v,
             vals_v, ybuf, sem, sem_g):
    wid = lax.axis_index("s") * 2 + lax.axis_index("c")
    lanes = lax.iota(jnp.int32, _L)
    lane_base = lanes * _NBINS
    ones_i = jnp.ones((_L,), jnp.int32)
    zeros_i = jnp.zeros((_L,), jnp.int32)
    neg2 = jnp.full((_L,), -2.0, jnp.float32)

    def rload(v):
        s = v // 8
        return row_v[s, pl.ds((v - s * 8) * _L, _L)]

    def cval(v):
        return cand_val[pl.ds(v * _L, _L)]

    def cidx(v):
        return cand_idx[pl.ds(v * _L, _L)]

    pltpu.async_copy(sims_hbm.at[wid * _ROWS_PER_W], row_v, sem)

    @pl.loop(0, _ROWS_PER_W)
    def _row(rl):
        r = wid * _ROWS_PER_W + rl
        pltpu.make_async_copy(sims_hbm.at[r], row_v, sem).wait()

        @pl.loop(0, _NBINS, unroll=8)
        def _zh(i):
            hist_v[pl.ds(i * _L, _L)] = zeros_i

        @pl.loop(0, _VREGS_ROW, unroll=10)
        def _pa(j):
            v = rload(j)
            b = (v * _SCALE + (0.5 - _LO * _SCALE)).astype(jnp.int32)
            plsc.addupdate_scatter(hist_v, [lane_base + b], ones_i)

        @pl.loop(0, _NBINS // _L, unroll=2)
        def _bs(c):
            acc = hist_v[pl.ds(c * _L, _L)]
            for l in range(1, _L):
                acc = acc + hist_v[pl.ds(l * _NBINS + c * _L, _L)]
            binsum_v[pl.ds(c * _L, _L)] = acc

        def _chunk_tot(c):
            return jnp.sum(binsum_v[pl.ds(c * _L, _L)])

        def _wcond(carry):
            cum, c = carry
            return jnp.logical_and(c > 0, cum + _chunk_tot(c) < _CHOOSE)

        def _wstep(carry):
            cum, c = carry
            return cum + _chunk_tot(c), c - 1

        cum, cstar = lax.while_loop(
            _wcond, _wstep, (jnp.int32(0), jnp.int32(_NBINS // _L - 1)))
        sfx = cum + plsc.cumsum(lax.rev(binsum_v[pl.ds(cstar * _L, _L)], (0,)))
        i = jnp.max(plsc.all_reduce_ffs(sfx >= _CHOOSE))
        bstar = cstar * _L + (_L - 1) - i
        tf = (bstar.astype(jnp.float32) - jnp.float32(0.5 - _LO * _SCALE)
              - 0.5) * jnp.float32(1.0 / _SCALE)
        tfv = jnp.broadcast_to(tf, (_L,))

        @pl.loop(0, _CVR)
        def _zc(i):
            cand_val[pl.ds(i * _L, _L)] = neg2
            cand_idx[pl.ds(i * _L, _L)] = zeros_i

        _G = 5

        def _pb(g, carry):
            off, jvec = carry
            vs = [rload(g * _G + k) for k in range(_G)]
            ms = [v >= tfv for v in vs]
            many = ms[0]
            for k in range(1, _G):
                many = jnp.logical_or(many, ms[k])

            def _scatter():
                o = off
                for k in range(_G):
                    pos = o + plsc.cumsum(ms[k].astype(jnp.int32)) - 1
                    gk = jnp.logical_and(ms[k], pos < _CAP)
                    plsc.store_scatter(cand_val, [pos], vs[k], mask=gk)
                    plsc.store_scatter(cand_idx, [pos], jvec + k * _L, mask=gk)
                    o = o + plsc.all_reduce_population_count(ms[k])
                return o

            new_off = lax.cond(jnp.any(many), _scatter, lambda: off)
            return new_off, jvec + _G * _L

        lax.fori_loop(0, _VREGS_ROW // _G, _pb, (zeros_i, lanes))

        @pl.when(rl + 1 < _ROWS_PER_W)
        def _pref():
            pltpu.async_copy(sims_hbm.at[r + 1], row_v, sem)

        def _ce(a, b, kv):
            ka = cval(a)
            kb = cval(b)
            ia = cidx(a)
            ib = cidx(b)
            desc = jnp.broadcast_to((a & kv) == 0, (_L,))
            swap = jnp.where(desc, ka < kb, ka > kb)
            cand_val[pl.ds(a * _L, _L)] = jnp.where(swap, kb, ka)
            cand_val[pl.ds(b * _L, _L)] = jnp.where(swap, ka, kb)
            cand_idx[pl.ds(a * _L, _L)] = jnp.where(swap, ib, ia)
            cand_idx[pl.ds(b * _L, _L)] = jnp.where(swap, ia, ib)

        def _vsort(v, desc):
            ks, xs = plsc.sort_key_val(cval(v), cidx(v), descending=desc)
            cand_val[pl.ds(v * _L, _L)] = ks
            cand_idx[pl.ds(v * _L, _L)] = xs

        @pl.loop(0, _CVR // 2)
        def _base(t):
            _vsort(2 * t, True)
            _vsort(2 * t + 1, False)

        for kv in (2, 4, 8, 16, 32):
            jv = kv // 2
            while jv >= 1:
                @pl.loop(0, _CVR // 2)
                def _stage(t, jv=jv, kv=kv):
                    blk = t // jv
                    a = blk * (2 * jv) + (t - blk * jv)
                    _ce(a, a + jv, kv)
                jv //= 2
            if kv < _CVR:
                @pl.loop(0, _CVR // 2)
                def _resort(t, kv=kv):
                    blk = t // kv
                    v = blk * (2 * kv) + (t - blk * kv)
                    _vsort(v, True)
                    _vsort(v + kv, False)
            else:
                @pl.loop(0, _CVR)
                def _resort_all(v):
                    _vsort(v, True)

        @pl.loop(0, _CHOOSE // _L)
        def _st(t):
            s = t // 8
            c = (t - s * 8) * _L
            osims_v[s, pl.ds(c, _L)] = cval(t)
            oidx_v[s, pl.ds(c, _L)] = cidx(t)

        cp0 = pltpu.async_copy(value_hbm.at[oidx_v.at[0]], vals_v.at[0], sem_g)
        cp1 = pltpu.async_copy(value_hbm.at[oidx_v.at[1]], vals_v.at[1], sem_g)
        pltpu.sync_copy(osims_v, out_sims.at[r])
        pltpu.sync_copy(oidx_v, out_idx.at[r])
        cp0.wait()
        cp1.wait()

        mx = jnp.max(cval(0))

        def _sm(t, carry):
            accn, accd = carry
            s = t // 8
            c = (t - s * 8) * _L
            e = jnp.exp((osims_v[s, pl.ds(c, _L)] - mx) * _INV_TEMP)
            return accn + e * vals_v[s, pl.ds(c, _L)], accd + e

        accn, accd = lax.fori_loop(
            0, _CHOOSE // _L, _sm,
            (jnp.zeros((_L,), jnp.float32), jnp.zeros((_L,), jnp.float32)))
        yv = (jnp.broadcast_to(jnp.sum(accn), (_L,))
              / jnp.broadcast_to(jnp.sum(accd), (_L,)))
        plsc.store_scatter(ybuf, [jnp.broadcast_to(rl, (_L,))],
                           yv, mask=lanes < 1)

    pltpu.sync_copy(ybuf, out_y.at[pl.ds(wid * _ROWS_PER_W, _ROWS_PER_W)])


_sc_select = pl.kernel(
    _sc_body,
    out_type=(
        jax.ShapeDtypeStruct((_B, 2, 128), jnp.float32),
        jax.ShapeDtypeStruct((_B, 2, 128), jnp.int32),
        jax.ShapeDtypeStruct((_B,), jnp.float32),
    ),
    mesh=plsc.VectorSubcoreMesh(core_axis_name="c", subcore_axis_name="s"),
    compiler_params=pltpu.CompilerParams(needs_layout_passes=False),
    scratch_types=[
        pltpu.VMEM((_SLAB, 128), jnp.float32),
        pltpu.VMEM((_NBINS * _L,), jnp.int32),
        pltpu.VMEM((_NBINS,), jnp.int32),
        pltpu.VMEM((_CAP,), jnp.float32),
        pltpu.VMEM((_CAP,), jnp.int32),
        pltpu.VMEM((2, 128), jnp.float32),
        pltpu.VMEM((2, 128), jnp.int32),
        pltpu.VMEM((2, 128), jnp.float32),
        pltpu.VMEM((_ROWS_PER_W,), jnp.float32),
        pltpu.SemaphoreType.DMA,
        pltpu.SemaphoreType.DMA,
    ],
)


def kernel(input, keys, value):
    nq = jnp.linalg.norm(input, axis=-1, keepdims=True) + 1e-8
    nk = jnp.linalg.norm(keys, axis=-1, keepdims=True) + 1e-8
    keys_pad = jnp.pad(keys, ((0, _NPAD - _N), (0, 0)))
    nk_pad = jnp.pad(nk, ((0, _NPAD - _N), (0, 0)), constant_values=1.0)
    sims = _sims_call(input, nq, keys_pad, nk_pad)
    topk_sims, topk_idx, y = _sc_select(sims, value)
    return (y, topk_sims.reshape(_B, _CHOOSE), topk_idx.reshape(_B, _CHOOSE))
